# baseline (speedup 1.0000x reference)
import functools

import jax
import jax.numpy as jnp
from jax import lax
from jax.experimental import pallas as pl
from jax.experimental.pallas import tpu as pltpu
from jax.experimental.pallas import tpu_sc as plsc

_NUM_SEG = 64
_EPS_PD = 1e-6
_INT_MAX = 2147483647


def _tc_colsum_body(nb, wf_ref, out_ref):
    i = pl.program_id(0)

    @pl.when(i == 0)
    def _():
        out_ref[...] = jnp.zeros_like(out_ref)

    out_ref[...] += jnp.sum(wf_ref[...], axis=0, keepdims=True)


def _tc_colsum_part(wf, rows, bn):
    n, d_model = wf.shape
    nb = rows // bn
    out = pl.pallas_call(
        functools.partial(_tc_colsum_body, nb),
        grid=(nb,),
        in_specs=[pl.BlockSpec((bn, d_model), lambda i: (i, 0))],
        out_specs=pl.BlockSpec((1, d_model), lambda i: (0, 0)),
        out_shape=jax.ShapeDtypeStruct((1, d_model), jnp.float32),
    )(wf)
    return out


def _tc_phase1_body(nb, bn, n, wf_ref, batch_ref, cs_ref, idx_ref, cb_ref,
                    colsum_ref, minval_ref, minidx_ref):
    i = pl.program_id(0)

    @pl.when(i == 0)
    def _():
        colsum_ref[...] = jnp.sum(cs_ref[...], axis=0, keepdims=True)
        minval_ref[...] = jnp.full_like(minval_ref, jnp.inf)
        minidx_ref[...] = jnp.full_like(minidx_ref, _INT_MAX)

    center = colsum_ref[...] / jnp.float32(n + 1e-8)
    diff = wf_ref[...] - center + _EPS_PD
    d2 = jnp.sum(diff * diff, axis=1, keepdims=True)
    seg = lax.broadcasted_iota(jnp.int32, (1, _NUM_SEG), 1)
    mask = batch_ref[...] == seg
    masked = jnp.where(mask, d2, jnp.float32(jnp.inf))
    bmin = jnp.min(masked, axis=0, keepdims=True)
    rowid = i * bn + lax.broadcasted_iota(jnp.int32, (bn, _NUM_SEG), 0)
    cand = jnp.where(mask & (masked == bmin), rowid, _INT_MAX)
    bidx = jnp.min(cand, axis=0, keepdims=True)
    better = bmin < minval_ref[...]
    minidx_ref[...] = jnp.where(better, bidx, minidx_ref[...])
    minval_ref[...] = jnp.minimum(minval_ref[...], bmin)

    @pl.when(i == nb - 1)
    def _():
        idx_ref[...] = jnp.clip(minidx_ref[...], 0, n - 1)
        last = jnp.max(batch_ref[...])
        segs = lax.broadcasted_iota(jnp.int32, (1, _NUM_SEG), 1)
        cb_ref[...] = jnp.where(minidx_ref[...] != _INT_MAX, segs, last)


def _tc_phase1(wf, batch2, cs, bn):
    n, d_model = wf.shape
    nb = n // bn
    idx, cb = pl.pallas_call(
        functools.partial(_tc_phase1_body, nb, bn, n),
        grid=(nb,),
        in_specs=[
            pl.BlockSpec((bn, d_model), lambda i: (i, 0)),
            pl.BlockSpec((bn, 1), lambda i: (i, 0)),
            pl.BlockSpec((32, d_model), lambda i: (0, 0)),
        ],
        out_specs=[
            pl.BlockSpec((1, _NUM_SEG), lambda i: (0, 0)),
            pl.BlockSpec((1, _NUM_SEG), lambda i: (0, 0)),
        ],
        out_shape=[
            jax.ShapeDtypeStruct((1, _NUM_SEG), jnp.int32),
            jax.ShapeDtypeStruct((1, _NUM_SEG), jnp.int32),
        ],
        scratch_shapes=[
            pltpu.VMEM((1, d_model), jnp.float32),
            pltpu.VMEM((1, _NUM_SEG), jnp.float32),
            pltpu.VMEM((1, _NUM_SEG), jnp.int32),
        ],
    )(wf, batch2, cs)
    return idx.reshape(_NUM_SEG), cb.reshape(_NUM_SEG)


def _sc_gather(idx, x):
    n, d_model = x.shape
    nw = 8
    rows_per = _NUM_SEG // nw
    mesh = plsc.VectorSubcoreMesh(core_axis_name="c", subcore_axis_name="s")

    @functools.partial(
        pl.kernel,
        mesh=mesh,
        out_type=jax.ShapeDtypeStruct((_NUM_SEG, d_model), jnp.float32),
        scratch_types=[
            pltpu.VMEM((rows_per,), jnp.int32),
            pltpu.VMEM((rows_per, d_model), jnp.float32),
            pltpu.SemaphoreType.DMA,
        ],
    )
    def gather(idx_hbm, x_hbm, out_hbm, idx_v, rows_v, sem):
        wid = lax.axis_index("s") * 2 + lax.axis_index("c")

        @pl.when(wid < nw)
        def _():
            base = wid * rows_per
            pltpu.sync_copy(idx_hbm.at[pl.ds(base, rows_per)], idx_v)
            pltpu.async_copy(x_hbm.at[idx_v], rows_v, sem).wait()
            pltpu.sync_copy(rows_v, out_hbm.at[pl.ds(base, rows_per)])

    return gather(idx, x)


def _sc_colsum(wf, row0, rows_total, nw, ch):
    n, d_model = wf.shape
    m = rows_total // nw
    nch = m // ch
    nvec = d_model // 16
    mesh = plsc.VectorSubcoreMesh(core_axis_name="c", subcore_axis_name="s")

    @functools.partial(
        pl.kernel,
        mesh=mesh,
        out_type=jax.ShapeDtypeStruct((nw * d_model,), jnp.float32),
        scratch_types=[
            pltpu.VMEM((2, ch, d_model), jnp.float32),
            pltpu.VMEM((d_model,), jnp.float32),
            pltpu.SemaphoreType.DMA,
            pltpu.SemaphoreType.DMA,
        ],
    )
    def colsum(wf_hbm, out_hbm, buf_v, acc_v, sem0, sem1):
        wid = lax.axis_index("s") * 2 + lax.axis_index("c")

        @pl.when(wid < nw)
        def _():
            base = row0 + wid * m
            sems = (sem0, sem1)
            handles = [None, None]
            handles[0] = pltpu.async_copy(
                wf_hbm.at[pl.ds(base, ch)], buf_v.at[0], sem0)
            acc = tuple(jnp.zeros((16,), jnp.float32) for _ in range(nvec))
            for k in range(nch):
                b = k % 2
                handles[b].wait()
                if k + 1 < nch:
                    nxt = (k + 1) % 2
                    handles[nxt] = pltpu.async_copy(
                        wf_hbm.at[pl.ds(base + (k + 1) * ch, ch)],
                        buf_v.at[nxt], sems[nxt])

                def row_body(r, carry, b=b):
                    return tuple(
                        carry[c] + buf_v[b, r, pl.ds(c * 16, 16)]
                        for c in range(nvec))

                acc = lax.fori_loop(0, ch, row_body, acc)
            for c in range(nvec):
                acc_v[pl.ds(c * 16, 16)] = acc[c]
            pltpu.sync_copy(acc_v, out_hbm.at[pl.ds(wid * d_model, d_model)])

    return colsum(wf)


_TC_ROWS = 34000
_SC_ROWS = 16000
_SC_NW = 25
_SC_CH = 40


def kernel(x, weighted_features, batch, mask_idx):
    n, d_model = weighted_features.shape
    batch2 = batch.reshape(n, 1)
    cs_tc = _tc_colsum_part(weighted_features, _TC_ROWS, 6800)
    cs_sc = _sc_colsum(weighted_features, _TC_ROWS, _SC_ROWS, _SC_NW, _SC_CH)
    cs = jnp.concatenate(
        [cs_tc, cs_sc.reshape(_SC_NW, d_model),
         jnp.zeros((32 - 1 - _SC_NW, d_model), jnp.float32)], axis=0)
    idx, cb = _tc_phase1(weighted_features, batch2, cs, 2000)
    centers = _sc_gather(idx, x)
    return centers, cb


# speedup vs baseline: 1.0570x; 1.0570x over previous
import functools

import jax
import jax.numpy as jnp
from jax import lax
from jax.experimental import pallas as pl
from jax.experimental.pallas import tpu as pltpu
from jax.experimental.pallas import tpu_sc as plsc

_NUM_SEG = 64
_EPS_PD = 1e-6
_INT_MAX = 2147483647


def _tc_colsum_body(nb, wf_ref, out_ref):
    i = pl.program_id(0)

    @pl.when(i == 0)
    def _():
        out_ref[...] = jnp.zeros_like(out_ref)

    out_ref[...] += jnp.sum(wf_ref[...], axis=0, keepdims=True)


def _tc_colsum_part(wf, rows, bn):
    n, d_model = wf.shape
    nb = rows // bn
    out = pl.pallas_call(
        functools.partial(_tc_colsum_body, nb),
        grid=(nb,),
        in_specs=[pl.BlockSpec((bn, d_model), lambda i: (i, 0))],
        out_specs=pl.BlockSpec((1, d_model), lambda i: (0, 0)),
        out_shape=jax.ShapeDtypeStruct((1, d_model), jnp.float32),
    )(wf)
    return out


def _tc_phase1_body(nb, bn, n, wf_ref, batch_ref, cs_ref, idx_ref, cb_ref,
                    colsum_ref, minval_ref, minidx_ref):
    i = pl.program_id(0)

    @pl.when(i == 0)
    def _():
        colsum_ref[...] = jnp.sum(cs_ref[...], axis=0, keepdims=True)
        minval_ref[...] = jnp.full_like(minval_ref, jnp.inf)
        minidx_ref[...] = jnp.full_like(minidx_ref, _INT_MAX)

    center = colsum_ref[...] / jnp.float32(n + 1e-8)
    diff = wf_ref[...] - center + _EPS_PD
    d2 = jnp.sum(diff * diff, axis=1, keepdims=True)
    seg = lax.broadcasted_iota(jnp.int32, (1, _NUM_SEG), 1)
    mask = batch_ref[...] == seg
    masked = jnp.where(mask, d2, jnp.float32(jnp.inf))
    bmin = jnp.min(masked, axis=0, keepdims=True)
    rowid = i * bn + lax.broadcasted_iota(jnp.int32, (bn, _NUM_SEG), 0)
    cand = jnp.where(mask & (masked == bmin), rowid, _INT_MAX)
    bidx = jnp.min(cand, axis=0, keepdims=True)
    better = bmin < minval_ref[...]
    minidx_ref[...] = jnp.where(better, bidx, minidx_ref[...])
    minval_ref[...] = jnp.minimum(minval_ref[...], bmin)

    @pl.when(i == nb - 1)
    def _():
        idx_ref[...] = jnp.clip(minidx_ref[...], 0, n - 1)
        last = jnp.max(batch_ref[...])
        segs = lax.broadcasted_iota(jnp.int32, (1, _NUM_SEG), 1)
        cb_ref[...] = jnp.where(minidx_ref[...] != _INT_MAX, segs, last)


def _tc_phase1(wf, batch2, cs, bn):
    n, d_model = wf.shape
    nb = n // bn
    idx, cb = pl.pallas_call(
        functools.partial(_tc_phase1_body, nb, bn, n),
        grid=(nb,),
        in_specs=[
            pl.BlockSpec((bn, d_model), lambda i: (i, 0)),
            pl.BlockSpec((bn, 1), lambda i: (i, 0)),
            pl.BlockSpec((40, d_model), lambda i: (0, 0)),
        ],
        out_specs=[
            pl.BlockSpec((1, _NUM_SEG), lambda i: (0, 0)),
            pl.BlockSpec((1, _NUM_SEG), lambda i: (0, 0)),
        ],
        out_shape=[
            jax.ShapeDtypeStruct((1, _NUM_SEG), jnp.int32),
            jax.ShapeDtypeStruct((1, _NUM_SEG), jnp.int32),
        ],
        scratch_shapes=[
            pltpu.VMEM((1, d_model), jnp.float32),
            pltpu.VMEM((1, _NUM_SEG), jnp.float32),
            pltpu.VMEM((1, _NUM_SEG), jnp.int32),
        ],
    )(wf, batch2, cs)
    return idx.reshape(_NUM_SEG), cb.reshape(_NUM_SEG)


def _sc_gather(idx, x):
    n, d_model = x.shape
    nw = 8
    rows_per = _NUM_SEG // nw
    mesh = plsc.VectorSubcoreMesh(core_axis_name="c", subcore_axis_name="s")

    @functools.partial(
        pl.kernel,
        mesh=mesh,
        out_type=jax.ShapeDtypeStruct((_NUM_SEG, d_model), jnp.float32),
        scratch_types=[
            pltpu.VMEM((rows_per,), jnp.int32),
            pltpu.VMEM((rows_per, d_model), jnp.float32),
            pltpu.SemaphoreType.DMA,
        ],
    )
    def gather(idx_hbm, x_hbm, out_hbm, idx_v, rows_v, sem):
        wid = lax.axis_index("s") * 2 + lax.axis_index("c")

        @pl.when(wid < nw)
        def _():
            base = wid * rows_per
            pltpu.sync_copy(idx_hbm.at[pl.ds(base, rows_per)], idx_v)
            pltpu.async_copy(x_hbm.at[idx_v], rows_v, sem).wait()
            pltpu.sync_copy(rows_v, out_hbm.at[pl.ds(base, rows_per)])

    return gather(idx, x)


def _sc_colsum(wf, row0, rows_total, nw, ch):
    n, d_model = wf.shape
    m = rows_total // nw
    nch = m // ch
    nvec = d_model // 16
    mesh = plsc.VectorSubcoreMesh(core_axis_name="c", subcore_axis_name="s")

    @functools.partial(
        pl.kernel,
        mesh=mesh,
        out_type=jax.ShapeDtypeStruct((nw * d_model,), jnp.float32),
        scratch_types=[
            pltpu.VMEM((2, ch, d_model), jnp.float32),
            pltpu.VMEM((d_model,), jnp.float32),
            pltpu.SemaphoreType.DMA,
            pltpu.SemaphoreType.DMA,
        ],
    )
    def colsum(wf_hbm, out_hbm, buf_v, acc_v, sem0, sem1):
        wid = lax.axis_index("s") * 2 + lax.axis_index("c")

        @pl.when(wid < nw)
        def _():
            base = row0 + wid * m
            sems = (sem0, sem1)
            handles = [None, None]
            handles[0] = pltpu.async_copy(
                wf_hbm.at[pl.ds(base, ch)], buf_v.at[0], sem0)
            acc = tuple(jnp.zeros((16,), jnp.float32) for _ in range(nvec))
            for k in range(nch):
                b = k % 2
                handles[b].wait()
                if k + 1 < nch:
                    nxt = (k + 1) % 2
                    handles[nxt] = pltpu.async_copy(
                        wf_hbm.at[pl.ds(base + (k + 1) * ch, ch)],
                        buf_v.at[nxt], sems[nxt])

                def row_body(r, carry, b=b):
                    return tuple(
                        carry[c] + buf_v[b, r, pl.ds(c * 16, 16)]
                        for c in range(nvec))

                acc = lax.fori_loop(0, ch, row_body, acc)
            for c in range(nvec):
                acc_v[pl.ds(c * 16, 16)] = acc[c]
            pltpu.sync_copy(acc_v, out_hbm.at[pl.ds(wid * d_model, d_model)])

    return colsum(wf)


_TC_ROWS = 30800
_TC_BN0 = 6160
_SC_ROWS = 19200
_SC_NW = 32
_SC_CH = 40


def kernel(x, weighted_features, batch, mask_idx):
    n, d_model = weighted_features.shape
    batch2 = batch.reshape(n, 1)
    cs_tc = _tc_colsum_part(weighted_features, _TC_ROWS, _TC_BN0)
    cs_sc = _sc_colsum(weighted_features, _TC_ROWS, _SC_ROWS, _SC_NW, _SC_CH)
    cs = jnp.concatenate(
        [cs_tc, cs_sc.reshape(_SC_NW, d_model),
         jnp.zeros((7, d_model), jnp.float32)], axis=0)
    idx, cb = _tc_phase1(weighted_features, batch2, cs, 5000)
    centers = _sc_gather(idx, x)
    return centers, cb
